# Initial kernel scaffold; baseline (speedup 1.0000x reference)
import jax
import jax.numpy as jnp
from jax.experimental import pallas as pl


def kernel(inputs, pos_embedding):
    raise NotImplementedError("write your pallas kernel here")


# speedup vs baseline: 3.4191x; 3.4191x over previous
import jax
import jax.numpy as jnp
from jax.experimental import pallas as pl


def _bcast_copy_body(table_ref, out_ref):
    out_ref[...] = table_ref[...][None]


def kernel(inputs, pos_embedding):
    B, S = inputs.shape
    P, D = pos_embedding.shape
    BS = 512
    grid = (S // BS, B)
    out = pl.pallas_call(
        _bcast_copy_body,
        grid=grid,
        in_specs=[pl.BlockSpec((BS, D), lambda i, j: (i, 0))],
        out_specs=pl.BlockSpec((1, BS, D), lambda i, j: (j, i, 0)),
        out_shape=jax.ShapeDtypeStruct((B, S, D), pos_embedding.dtype),
    )(pos_embedding)
    return out


# speedup vs baseline: 4.1832x; 1.2235x over previous
import jax
import jax.numpy as jnp
from jax.experimental import pallas as pl


def _bcast_copy_body(table_ref, out_ref):
    out_ref[...] = table_ref[...][None]


def kernel(inputs, pos_embedding):
    B, S = inputs.shape
    P, D = pos_embedding.shape
    BS = 1024
    grid = (S // BS, B)
    out = pl.pallas_call(
        _bcast_copy_body,
        grid=grid,
        in_specs=[pl.BlockSpec((BS, D), lambda i, j: (i, 0))],
        out_specs=pl.BlockSpec((1, BS, D), lambda i, j: (j, i, 0)),
        out_shape=jax.ShapeDtypeStruct((B, S, D), pos_embedding.dtype),
    )(pos_embedding)
    return out


# speedup vs baseline: 4.5941x; 1.0982x over previous
import jax
import jax.numpy as jnp
from jax.experimental import pallas as pl


def _bcast_copy_body(table_ref, out_ref):
    out_ref[...] = table_ref[...][None]


def kernel(inputs, pos_embedding):
    B, S = inputs.shape
    P, D = pos_embedding.shape
    BS = 2048
    grid = (S // BS, B)
    out = pl.pallas_call(
        _bcast_copy_body,
        grid=grid,
        in_specs=[pl.BlockSpec((BS, D), lambda i, j: (i, 0))],
        out_specs=pl.BlockSpec((1, BS, D), lambda i, j: (j, i, 0)),
        out_shape=jax.ShapeDtypeStruct((B, S, D), pos_embedding.dtype),
    )(pos_embedding)
    return out


# speedup vs baseline: 5.1794x; 1.1274x over previous
import jax
import jax.numpy as jnp
from jax.experimental import pallas as pl
from jax.experimental.pallas import tpu as pltpu


def _bcast_copy_body(table_ref, out_ref):
    out_ref[...] = jnp.broadcast_to(table_ref[...][None], out_ref.shape)


def kernel(inputs, pos_embedding):
    B, S = inputs.shape
    P, D = pos_embedding.shape
    BS = 1024
    grid = (S // BS,)
    out = pl.pallas_call(
        _bcast_copy_body,
        grid=grid,
        in_specs=[pl.BlockSpec((BS, D), lambda i: (i, 0))],
        out_specs=pl.BlockSpec((B, BS, D), lambda i: (0, i, 0)),
        out_shape=jax.ShapeDtypeStruct((B, S, D), pos_embedding.dtype),
        compiler_params=pltpu.CompilerParams(vmem_limit_bytes=63 * 1024 * 1024),
    )(pos_embedding)
    return out
